# baseline (speedup 1.0000x reference)
import functools

import jax
import jax.numpy as jnp
from jax import lax
from jax.experimental import pallas as pl
from jax.experimental.pallas import tpu as pltpu
from jax.experimental.pallas import tpu_sc as plsc

N = 10000
E = 320000
D = 128

NC = 2
NS = 16
NW = NC * NS
ROWS = 320
NPAD = NW * ROWS

BLK = 4000
NBLK = E // BLK
CHUNKS = BLK // 16
GS = 32

NEG_INF = float("-inf")


def _sc_aggregate(feats, src, dst):
  mesh = plsc.VectorSubcoreMesh(core_axis_name="c", subcore_axis_name="s")

  @functools.partial(
      pl.kernel,
      out_type=jax.ShapeDtypeStruct((NPAD, D), jnp.float32),
      mesh=mesh,
      scratch_types=[
          pltpu.VMEM((ROWS + 1, D), jnp.float32),
          pltpu.VMEM((ROWS + 1, D), jnp.float32),
          pltpu.VMEM((BLK,), jnp.int32),
          pltpu.VMEM((BLK,), jnp.int32),
          pltpu.VMEM((BLK,), jnp.int32),
          pltpu.VMEM((BLK,), jnp.int32),
          pltpu.VMEM((BLK + 64,), jnp.int32),
          pltpu.VMEM((BLK + 64,), jnp.int32),
          pltpu.VMEM((2, GS, D), jnp.float32),
          pltpu.SemaphoreType.DMA,
          pltpu.SemaphoreType.DMA,
          pltpu.SemaphoreType.DMA,
          pltpu.SemaphoreType.DMA,
      ],
      compiler_params=pltpu.CompilerParams(needs_layout_passes=False),
  )
  def sc_kernel(feats_hbm, src_hbm, dst_hbm, out_hbm,
                acc0, acc1, sbuf0, sbuf1, dbuf0, dbuf1, lsrc, ldst, gring,
                gs0, gs1, bs0, bs1):
    wid = lax.axis_index("s") * NC + lax.axis_index("c")
    lo = wid * ROWS
    hi = lo + ROWS

    @pl.loop(0, ROWS + 1)
    def _(r):
      for q in range(D // 16):
        neg = jnp.full((16,), NEG_INF, jnp.float32)
        acc0[r, pl.ds(q * 16, 16)] = neg
        acc1[r, pl.ds(q * 16, 16)] = neg

    def fire_block(b, sb, db, sem):
      pltpu.make_async_copy(src_hbm.at[pl.ds(b * BLK, BLK)], sb, sem).start()
      pltpu.make_async_copy(dst_hbm.at[pl.ds(b * BLK, BLK)], db, sem).start()

    def wait_block(b, sb, db, sem):
      pltpu.make_async_copy(src_hbm.at[pl.ds(b * BLK, BLK)], sb, sem).wait()
      pltpu.make_async_copy(dst_hbm.at[pl.ds(b * BLK, BLK)], db, sem).wait()

    def fire32(slot, sem, g):
      pltpu.make_async_copy(feats_hbm.at[lsrc.at[pl.ds(g * GS, GS)]],
                            gring.at[slot], sem).start()

    def drain32(slot, sem, g):
      pltpu.make_async_copy(feats_hbm.at[lsrc.at[pl.ds(g * GS, GS)]],
                            gring.at[slot], sem).wait()

    def process32(slot, g):
      for h in range(GS // 16):
        dv = ldst[pl.ds(g * GS + h * 16, 16)]
        for r in range(16):
          dl = dv[r]
          acc = acc0 if r % 2 == 0 else acc1
          for q in range(D // 16):
            sl = pl.ds(q * 16, 16)
            acc[dl, sl] = jnp.maximum(acc[dl, sl],
                                      gring[slot, h * 16 + r, sl])

    def do_block(b, sb, db, nsb, ndb, nsem, cnt_in):
      @pl.when(b + 1 < NBLK)
      def _():
        fire_block(b + 1, nsb, ndb, nsem)

      def filt(c, cv):
        d = db[pl.ds(c * 16, 16)]
        m = (d >= lo) & (d < hi)
        cur = cv[0]
        s = sb[pl.ds(c * 16, 16)]
        plsc.store_compressed(lsrc.at[pl.ds(cur, 16)], s, mask=m)
        plsc.store_compressed(ldst.at[pl.ds(cur, 16)], d - lo, mask=m)
        return cv + plsc.all_reduce_population_count(m)

      cv_fin = plsc.parallel_loop(
          0, CHUNKS, 1, unroll=8,
          carry=jnp.broadcast_to(cnt_in, (16,)))(filt)
      cnt = cv_fin[0]
      gfull = cnt // GS

      @pl.when(gfull >= 1)
      def _():
        fire32(0, gs0, 0)

      @pl.when(gfull >= 2)
      def _():
        fire32(1, gs1, 1)

      def pair(i, carry):
        g0 = 2 * i
        drain32(0, gs0, g0)
        process32(0, g0)

        @pl.when(g0 + 2 < gfull)
        def _():
          fire32(0, gs0, g0 + 2)

        drain32(1, gs1, g0 + 1)
        process32(1, g0 + 1)

        @pl.when(g0 + 3 < gfull)
        def _():
          fire32(1, gs1, g0 + 3)

        return carry

      lax.fori_loop(0, gfull // 2, pair, jnp.int32(0))

      @pl.when(gfull % 2 == 1)
      def _():
        drain32(0, gs0, gfull - 1)
        process32(0, gfull - 1)

      rem = cnt - gfull * GS
      base = gfull * GS
      for t in range(GS // 16):
        svec = lsrc[pl.ds(base + 16 * t, 16)]
        dvec = ldst[pl.ds(base + 16 * t, 16)]
        lsrc[pl.ds(16 * t, 16)] = svec
        ldst[pl.ds(16 * t, 16)] = dvec
      return rem

    fire_block(0, sbuf0, dbuf0, bs0)

    def two_blocks(j, cnt):
      b = 2 * j
      wait_block(b, sbuf0, dbuf0, bs0)
      cnt = do_block(b, sbuf0, dbuf0, sbuf1, dbuf1, bs1, cnt)
      wait_block(b + 1, sbuf1, dbuf1, bs1)
      cnt = do_block(b + 1, sbuf1, dbuf1, sbuf0, dbuf0, bs0, cnt)
      return cnt

    cnt = lax.fori_loop(0, NBLK // 2, two_blocks, jnp.int32(0))

    @pl.when(cnt > 0)
    def _():
      pad_s = jnp.broadcast_to(lo, (16,))
      pad_d = jnp.full((16,), ROWS, jnp.int32)
      for t in range(GS // 16):
        lsrc[pl.ds(cnt + 16 * t, 16)] = pad_s
        ldst[pl.ds(cnt + 16 * t, 16)] = pad_d
      fire32(0, gs0, 0)
      drain32(0, gs0, 0)
      process32(0, 0)

    @pl.loop(0, ROWS)
    def _(r):
      for q in range(D // 16):
        sl = pl.ds(q * 16, 16)
        acc0[r, sl] = jnp.maximum(acc0[r, sl], acc1[r, sl])

    pltpu.sync_copy(acc0.at[pl.ds(0, ROWS)], out_hbm.at[pl.ds(lo, ROWS)])

  return sc_kernel(feats, src, dst)


def _tc_mlp(feats, agg, w1t, b1, g1, be1, w2t, b2, g2, be2, g3, be3):
  def body(feats_ref, agg_ref, w1t_ref, b1_ref, g1_ref, be1_ref,
           w2t_ref, b2_ref, g2_ref, be2_ref, g3_ref, be3_ref, out_ref):
    a = agg_ref[...]
    a = jnp.where(a == NEG_INF, 0.0, a)
    h = feats_ref[...] + a

    def bn(x, gamma, beta):
      mu = jnp.mean(x, axis=0, keepdims=True)
      var = jnp.mean(x * x, axis=0, keepdims=True) - mu * mu
      return (x - mu) * lax.rsqrt(var + 1e-5) * gamma + beta

    h = jnp.dot(h, w1t_ref[...], preferred_element_type=jnp.float32)
    h = bn(h + b1_ref[...], g1_ref[...], be1_ref[...])
    h = jnp.maximum(h, 0.0)
    h = jnp.dot(h, w2t_ref[...], preferred_element_type=jnp.float32)
    h = bn(h + b2_ref[...], g2_ref[...], be2_ref[...])
    h = jnp.maximum(h, 0.0)
    out_ref[...] = bn(h, g3_ref[...], be3_ref[...])

  vecs = [b1, g1, be1, b2, g2, be2, g3, be3]
  vecs2d = [v.reshape(1, D) for v in vecs]
  return pl.pallas_call(
      body,
      out_shape=jax.ShapeDtypeStruct((N, D), jnp.float32),
  )(feats, agg, w1t, vecs2d[0], vecs2d[1], vecs2d[2],
    w2t, vecs2d[3], vecs2d[4], vecs2d[5], vecs2d[6], vecs2d[7])


@jax.jit
def kernel(feats, edge_index, W1, b1, g1, be1, W2, b2, g2, be2, g3, be3):
  src = edge_index[0]
  dst = edge_index[1]
  agg = _sc_aggregate(feats, src, dst)[:N]
  return _tc_mlp(feats, agg, W1.T, b1, g1, be1, W2.T, b2, g2, be2, g3, be3)


# speedup vs baseline: 1.0044x; 1.0044x over previous
import functools

import jax
import jax.numpy as jnp
from jax import lax
from jax.experimental import pallas as pl
from jax.experimental.pallas import tpu as pltpu
from jax.experimental.pallas import tpu_sc as plsc

N = 10000
E = 320000
D = 128

NC = 2
NS = 16
NW = NC * NS
ROWS = 320
NPAD = NW * ROWS

BLK = 4000
NBLK = E // BLK
CHUNKS = BLK // 16
GS = 32

NEG_INF = float("-inf")


def _sc_aggregate(feats, src, dst):
  mesh = plsc.VectorSubcoreMesh(core_axis_name="c", subcore_axis_name="s")

  @functools.partial(
      pl.kernel,
      out_type=jax.ShapeDtypeStruct((NPAD, D), jnp.float32),
      mesh=mesh,
      scratch_types=[
          pltpu.VMEM((ROWS + 1, D), jnp.float32),
          pltpu.VMEM((BLK,), jnp.int32),
          pltpu.VMEM((BLK,), jnp.int32),
          pltpu.VMEM((BLK,), jnp.int32),
          pltpu.VMEM((BLK,), jnp.int32),
          pltpu.VMEM((BLK + 64,), jnp.int32),
          pltpu.VMEM((BLK + 64,), jnp.int32),
          pltpu.VMEM((2, GS, D), jnp.float32),
          pltpu.SemaphoreType.DMA,
          pltpu.SemaphoreType.DMA,
          pltpu.SemaphoreType.DMA,
          pltpu.SemaphoreType.DMA,
      ],
      compiler_params=pltpu.CompilerParams(needs_layout_passes=False),
  )
  def sc_kernel(feats_hbm, src_hbm, dst_hbm, out_hbm,
                acc, sbuf0, sbuf1, dbuf0, dbuf1, lsrc, ldst, gring,
                gs0, gs1, bs0, bs1):
    wid = lax.axis_index("s") * NC + lax.axis_index("c")
    lo = wid * ROWS
    hi = lo + ROWS

    @pl.loop(0, ROWS + 1)
    def _(r):
      for q in range(D // 16):
        acc[r, pl.ds(q * 16, 16)] = jnp.full((16,), NEG_INF, jnp.float32)

    def fire_block(b, sb, db, sem):
      pltpu.make_async_copy(src_hbm.at[pl.ds(b * BLK, BLK)], sb, sem).start()
      pltpu.make_async_copy(dst_hbm.at[pl.ds(b * BLK, BLK)], db, sem).start()

    def wait_block(b, sb, db, sem):
      pltpu.make_async_copy(src_hbm.at[pl.ds(b * BLK, BLK)], sb, sem).wait()
      pltpu.make_async_copy(dst_hbm.at[pl.ds(b * BLK, BLK)], db, sem).wait()

    def fire32(slot, sem, g):
      pltpu.make_async_copy(feats_hbm.at[lsrc.at[pl.ds(g * GS, GS)]],
                            gring.at[slot], sem).start()

    def drain32(slot, sem, g):
      pltpu.make_async_copy(feats_hbm.at[lsrc.at[pl.ds(g * GS, GS)]],
                            gring.at[slot], sem).wait()

    def process32(slot, g):
      for h in range(GS // 16):
        dv = ldst[pl.ds(g * GS + h * 16, 16)]
        for r in range(16):
          dl = dv[r]
          for q in range(D // 16):
            sl = pl.ds(q * 16, 16)
            acc[dl, sl] = jnp.maximum(acc[dl, sl],
                                      gring[slot, h * 16 + r, sl])

    def do_block(b, sb, db, nsb, ndb, nsem, cnt_in):
      @pl.when(b + 1 < NBLK)
      def _():
        fire_block(b + 1, nsb, ndb, nsem)

      def filt(c, cv):
        d = db[pl.ds(c * 16, 16)]
        m = (d >= lo) & (d < hi)
        cur = cv[0]
        s = sb[pl.ds(c * 16, 16)]
        plsc.store_compressed(lsrc.at[pl.ds(cur, 16)], s, mask=m)
        plsc.store_compressed(ldst.at[pl.ds(cur, 16)], d - lo, mask=m)
        return cv + plsc.all_reduce_population_count(m)

      cv_fin = plsc.parallel_loop(
          0, CHUNKS, 1, unroll=8,
          carry=jnp.broadcast_to(cnt_in, (16,)))(filt)
      cnt = cv_fin[0]
      gfull = cnt // GS

      @pl.when(gfull >= 1)
      def _():
        fire32(0, gs0, 0)

      @pl.when(gfull >= 2)
      def _():
        fire32(1, gs1, 1)

      def pair(i, carry):
        g0 = 2 * i
        drain32(0, gs0, g0)
        process32(0, g0)

        @pl.when(g0 + 2 < gfull)
        def _():
          fire32(0, gs0, g0 + 2)

        drain32(1, gs1, g0 + 1)
        process32(1, g0 + 1)

        @pl.when(g0 + 3 < gfull)
        def _():
          fire32(1, gs1, g0 + 3)

        return carry

      lax.fori_loop(0, gfull // 2, pair, jnp.int32(0))

      @pl.when(gfull % 2 == 1)
      def _():
        drain32(0, gs0, gfull - 1)
        process32(0, gfull - 1)

      rem = cnt - gfull * GS
      base = gfull * GS
      for t in range(GS // 16):
        svec = lsrc[pl.ds(base + 16 * t, 16)]
        dvec = ldst[pl.ds(base + 16 * t, 16)]
        lsrc[pl.ds(16 * t, 16)] = svec
        ldst[pl.ds(16 * t, 16)] = dvec
      return rem

    fire_block(0, sbuf0, dbuf0, bs0)

    def two_blocks(j, cnt):
      b = 2 * j
      wait_block(b, sbuf0, dbuf0, bs0)
      cnt = do_block(b, sbuf0, dbuf0, sbuf1, dbuf1, bs1, cnt)
      wait_block(b + 1, sbuf1, dbuf1, bs1)
      cnt = do_block(b + 1, sbuf1, dbuf1, sbuf0, dbuf0, bs0, cnt)
      return cnt

    cnt = lax.fori_loop(0, NBLK // 2, two_blocks, jnp.int32(0))

    @pl.when(cnt > 0)
    def _():
      pad_s = jnp.broadcast_to(lo, (16,))
      pad_d = jnp.full((16,), ROWS, jnp.int32)
      for t in range(GS // 16):
        lsrc[pl.ds(cnt + 16 * t, 16)] = pad_s
        ldst[pl.ds(cnt + 16 * t, 16)] = pad_d
      fire32(0, gs0, 0)
      drain32(0, gs0, 0)
      process32(0, 0)

    pltpu.sync_copy(acc.at[pl.ds(0, ROWS)], out_hbm.at[pl.ds(lo, ROWS)])

  return sc_kernel(feats, src, dst)


def _tc_mlp(feats, agg, w1t, b1, g1, be1, w2t, b2, g2, be2, g3, be3):
  def body(feats_ref, agg_ref, w1t_ref, b1_ref, g1_ref, be1_ref,
           w2t_ref, b2_ref, g2_ref, be2_ref, g3_ref, be3_ref, out_ref):
    a = agg_ref[...]
    a = jnp.where(a == NEG_INF, 0.0, a)
    h = feats_ref[...] + a

    def bn(x, gamma, beta):
      mu = jnp.mean(x, axis=0, keepdims=True)
      var = jnp.mean(x * x, axis=0, keepdims=True) - mu * mu
      return (x - mu) * lax.rsqrt(var + 1e-5) * gamma + beta

    h = jnp.dot(h, w1t_ref[...], preferred_element_type=jnp.float32)
    h = bn(h + b1_ref[...], g1_ref[...], be1_ref[...])
    h = jnp.maximum(h, 0.0)
    h = jnp.dot(h, w2t_ref[...], preferred_element_type=jnp.float32)
    h = bn(h + b2_ref[...], g2_ref[...], be2_ref[...])
    h = jnp.maximum(h, 0.0)
    out_ref[...] = bn(h, g3_ref[...], be3_ref[...])

  vecs = [b1, g1, be1, b2, g2, be2, g3, be3]
  vecs2d = [v.reshape(1, D) for v in vecs]
  return pl.pallas_call(
      body,
      out_shape=jax.ShapeDtypeStruct((N, D), jnp.float32),
  )(feats, agg, w1t, vecs2d[0], vecs2d[1], vecs2d[2],
    w2t, vecs2d[3], vecs2d[4], vecs2d[5], vecs2d[6], vecs2d[7])


@jax.jit
def kernel(feats, edge_index, W1, b1, g1, be1, W2, b2, g2, be2, g3, be3):
  src = edge_index[0]
  dst = edge_index[1]
  agg = _sc_aggregate(feats, src, dst)[:N]
  return _tc_mlp(feats, agg, W1.T, b1, g1, be1, W2.T, b2, g2, be2, g3, be3)


# speedup vs baseline: 1.3457x; 1.3398x over previous
import functools

import jax
import jax.numpy as jnp
from jax import lax
from jax.experimental import pallas as pl
from jax.experimental.pallas import tpu as pltpu
from jax.experimental.pallas import tpu_sc as plsc

N = 10000
E = 320000
D = 128

NC = 2
NS = 16
NW = NC * NS
ROWS = 320
NPAD = NW * ROWS

BLK = 8000
NBLK = E // BLK
CHUNKS = BLK // 16
GS = 32

NEG_INF = float("-inf")


def _sc_aggregate(feats, src, dst):
  mesh = plsc.VectorSubcoreMesh(core_axis_name="c", subcore_axis_name="s")

  @functools.partial(
      pl.kernel,
      out_type=jax.ShapeDtypeStruct((NPAD, D), jnp.float32),
      mesh=mesh,
      scratch_types=[
          pltpu.VMEM((ROWS + 1, D), jnp.float32),
          pltpu.VMEM((BLK,), jnp.int32),
          pltpu.VMEM((BLK,), jnp.int32),
          pltpu.VMEM((BLK,), jnp.int32),
          pltpu.VMEM((BLK,), jnp.int32),
          pltpu.VMEM((BLK + 64,), jnp.int32),
          pltpu.VMEM((BLK + 64,), jnp.int32),
          pltpu.VMEM((2, GS, D), jnp.float32),
          pltpu.SemaphoreType.DMA,
          pltpu.SemaphoreType.DMA,
          pltpu.SemaphoreType.DMA,
          pltpu.SemaphoreType.DMA,
      ],
      compiler_params=pltpu.CompilerParams(needs_layout_passes=False),
  )
  def sc_kernel(feats_hbm, src_hbm, dst_hbm, out_hbm,
                acc, sbuf0, sbuf1, dbuf0, dbuf1, lsrc, ldst, gring,
                gs0, gs1, bs0, bs1):
    wid = lax.axis_index("s") * NC + lax.axis_index("c")
    lo = wid * ROWS
    hi = lo + ROWS

    @pl.loop(0, ROWS + 1)
    def _(r):
      for q in range(D // 16):
        acc[r, pl.ds(q * 16, 16)] = jnp.full((16,), NEG_INF, jnp.float32)

    def fire_block(b, sb, db, sem):
      pltpu.make_async_copy(src_hbm.at[pl.ds(b * BLK, BLK)], sb, sem).start()
      pltpu.make_async_copy(dst_hbm.at[pl.ds(b * BLK, BLK)], db, sem).start()

    def wait_block(b, sb, db, sem):
      pltpu.make_async_copy(src_hbm.at[pl.ds(b * BLK, BLK)], sb, sem).wait()
      pltpu.make_async_copy(dst_hbm.at[pl.ds(b * BLK, BLK)], db, sem).wait()

    def fire32(slot, sem, g):
      pltpu.make_async_copy(feats_hbm.at[lsrc.at[pl.ds(g * GS, GS)]],
                            gring.at[slot], sem).start()

    def drain32(slot, sem, g):
      pltpu.make_async_copy(feats_hbm.at[lsrc.at[pl.ds(g * GS, GS)]],
                            gring.at[slot], sem).wait()

    def process32(slot, g):
      for h in range(GS // 16):
        dv = ldst[pl.ds(g * GS + h * 16, 16)]
        for r in range(16):
          dl = dv[r]
          row = h * 16 + r
          avs = [acc[dl, pl.ds(q * 16, 16)] for q in range(D // 16)]
          mvs = [gring[slot, row, pl.ds(q * 16, 16)] for q in range(D // 16)]
          for q in range(D // 16):
            acc[dl, pl.ds(q * 16, 16)] = jnp.maximum(avs[q], mvs[q])

    def do_block(b, sb, db, nsb, ndb, nsem, cnt_in):
      @pl.when(b + 1 < NBLK)
      def _():
        fire_block(b + 1, nsb, ndb, nsem)

      def filt(c, cv):
        d = db[pl.ds(c * 16, 16)]
        m = (d >= lo) & (d < hi)
        cur = cv[0]
        s = sb[pl.ds(c * 16, 16)]
        plsc.store_compressed(lsrc.at[pl.ds(cur, 16)], s, mask=m)
        plsc.store_compressed(ldst.at[pl.ds(cur, 16)], d - lo, mask=m)
        return cv + plsc.all_reduce_population_count(m)

      cv_fin = plsc.parallel_loop(
          0, CHUNKS, 1, unroll=8,
          carry=jnp.broadcast_to(cnt_in, (16,)))(filt)
      cnt = cv_fin[0]
      gfull = cnt // GS

      @pl.when(gfull >= 1)
      def _():
        fire32(0, gs0, 0)

      @pl.when(gfull >= 2)
      def _():
        fire32(1, gs1, 1)

      def pair(i, carry):
        g0 = 2 * i
        drain32(0, gs0, g0)
        process32(0, g0)

        @pl.when(g0 + 2 < gfull)
        def _():
          fire32(0, gs0, g0 + 2)

        drain32(1, gs1, g0 + 1)
        process32(1, g0 + 1)

        @pl.when(g0 + 3 < gfull)
        def _():
          fire32(1, gs1, g0 + 3)

        return carry

      lax.fori_loop(0, gfull // 2, pair, jnp.int32(0))

      @pl.when(gfull % 2 == 1)
      def _():
        drain32(0, gs0, gfull - 1)
        process32(0, gfull - 1)

      rem = cnt - gfull * GS
      base = gfull * GS
      for t in range(GS // 16):
        svec = lsrc[pl.ds(base + 16 * t, 16)]
        dvec = ldst[pl.ds(base + 16 * t, 16)]
        lsrc[pl.ds(16 * t, 16)] = svec
        ldst[pl.ds(16 * t, 16)] = dvec
      return rem

    fire_block(0, sbuf0, dbuf0, bs0)

    def two_blocks(j, cnt):
      b = 2 * j
      wait_block(b, sbuf0, dbuf0, bs0)
      cnt = do_block(b, sbuf0, dbuf0, sbuf1, dbuf1, bs1, cnt)
      wait_block(b + 1, sbuf1, dbuf1, bs1)
      cnt = do_block(b + 1, sbuf1, dbuf1, sbuf0, dbuf0, bs0, cnt)
      return cnt

    cnt = lax.fori_loop(0, NBLK // 2, two_blocks, jnp.int32(0))

    @pl.when(cnt > 0)
    def _():
      pad_s = jnp.broadcast_to(lo, (16,))
      pad_d = jnp.full((16,), ROWS, jnp.int32)
      for t in range(GS // 16):
        lsrc[pl.ds(cnt + 16 * t, 16)] = pad_s
        ldst[pl.ds(cnt + 16 * t, 16)] = pad_d
      fire32(0, gs0, 0)
      drain32(0, gs0, 0)
      process32(0, 0)

    pltpu.sync_copy(acc.at[pl.ds(0, ROWS)], out_hbm.at[pl.ds(lo, ROWS)])

  return sc_kernel(feats, src, dst)


def _tc_mlp(feats, agg, w1t, b1, g1, be1, w2t, b2, g2, be2, g3, be3):
  def body(feats_ref, agg_ref, w1t_ref, b1_ref, g1_ref, be1_ref,
           w2t_ref, b2_ref, g2_ref, be2_ref, g3_ref, be3_ref, out_ref):
    a = agg_ref[...]
    a = jnp.where(a == NEG_INF, 0.0, a)
    h = feats_ref[...] + a

    def bn(x, gamma, beta):
      mu = jnp.mean(x, axis=0, keepdims=True)
      var = jnp.mean(x * x, axis=0, keepdims=True) - mu * mu
      return (x - mu) * lax.rsqrt(var + 1e-5) * gamma + beta

    h = jnp.dot(h, w1t_ref[...], preferred_element_type=jnp.float32)
    h = bn(h + b1_ref[...], g1_ref[...], be1_ref[...])
    h = jnp.maximum(h, 0.0)
    h = jnp.dot(h, w2t_ref[...], preferred_element_type=jnp.float32)
    h = bn(h + b2_ref[...], g2_ref[...], be2_ref[...])
    h = jnp.maximum(h, 0.0)
    out_ref[...] = bn(h, g3_ref[...], be3_ref[...])

  vecs = [b1, g1, be1, b2, g2, be2, g3, be3]
  vecs2d = [v.reshape(1, D) for v in vecs]
  return pl.pallas_call(
      body,
      out_shape=jax.ShapeDtypeStruct((N, D), jnp.float32),
  )(feats, agg, w1t, vecs2d[0], vecs2d[1], vecs2d[2],
    w2t, vecs2d[3], vecs2d[4], vecs2d[5], vecs2d[6], vecs2d[7])


@jax.jit
def kernel(feats, edge_index, W1, b1, g1, be1, W2, b2, g2, be2, g3, be3):
  src = edge_index[0]
  dst = edge_index[1]
  agg = _sc_aggregate(feats, src, dst)[:N]
  return _tc_mlp(feats, agg, W1.T, b1, g1, be1, W2.T, b2, g2, be2, g3, be3)
